# baseline (device time: 522632 ns/iter reference)
import jax
import jax.numpy as jnp
from jax import lax
from jax.experimental import pallas as pl
from jax.experimental.pallas import tpu as pltpu

M, D = 8192, 2048
EPS = 1e-6

K = 16
C = K // 2
ROWS = M // K
HALF = M // 2


def kernel(partial, resid, gamma):
    p2d = partial.reshape(M, D)
    gamma2d = gamma.reshape(1, D)

    def body(
        p_any,
        p_blk,
        r_blk,
        g_blk,
        o_blk,
        recv_any,
        stage,
        y_send_sems,
        y_recv_sems,
        x_send_sems,
        x_recv_sems,
        copy_sem,
    ):
        k = pl.program_id(0)
        my_x = lax.axis_index("x")
        my_y = lax.axis_index("y")
        my_z = lax.axis_index("z")
        ynbr = (my_x, 1 - my_y, my_z)
        xnbr = (1 - my_x, my_y, my_z)
        half0 = my_x * HALF

        def y_rdma(c):
            return pltpu.make_async_remote_copy(
                src_ref=p_any.at[0, pl.ds(half0 + c * ROWS, ROWS), :],
                dst_ref=recv_any.at[pl.ds(half0 + c * ROWS, ROWS), :],
                send_sem=y_send_sems.at[c],
                recv_sem=y_recv_sems.at[c],
                device_id=ynbr,
                device_id_type=pl.DeviceIdType.MESH,
            )

        def fwd_rdma(c):
            return pltpu.make_async_remote_copy(
                src_ref=recv_any.at[pl.ds(half0 + c * ROWS, ROWS), :],
                dst_ref=recv_any.at[pl.ds(half0 + c * ROWS, ROWS), :],
                send_sem=x_send_sems.at[c],
                recv_sem=x_recv_sems.at[c],
                device_id=xnbr,
                device_id_type=pl.DeviceIdType.MESH,
            )

        @pl.when(k == 0)
        def _():
            barrier = pltpu.get_barrier_semaphore()
            for nbr in (ynbr, xnbr):
                pl.semaphore_signal(
                    barrier, inc=1, device_id=nbr,
                    device_id_type=pl.DeviceIdType.MESH,
                )
            pl.semaphore_wait(barrier, 2)
            for c in range(C):
                y_rdma(c).start()

        @pl.when(k < C)
        def _():
            pltpu.make_async_remote_copy(
                src_ref=p_any.at[0, pl.ds(0, ROWS), :],
                dst_ref=recv_any.at[pl.ds(half0 + k * ROWS, ROWS), :],
                send_sem=y_send_sems.at[0],
                recv_sem=y_recv_sems.at[k],
                device_id=ynbr,
                device_id_type=pl.DeviceIdType.MESH,
            ).wait_recv()
            fwd_rdma(k).start()

        @pl.when(((my_x == 0) & (k >= C)) | ((my_x == 1) & (k < C)))
        def _():
            cc = k % C
            pltpu.make_async_remote_copy(
                src_ref=recv_any.at[pl.ds(0, ROWS), :],
                dst_ref=recv_any.at[
                    pl.ds((1 - my_x) * HALF + cc * ROWS, ROWS), :
                ],
                send_sem=x_send_sems.at[0],
                recv_sem=x_recv_sems.at[cc],
                device_id=xnbr,
                device_id_type=pl.DeviceIdType.MESH,
            ).wait_recv()

        cp = pltpu.make_async_copy(
            recv_any.at[pl.ds(k * ROWS, ROWS), :], stage, copy_sem
        )
        cp.start()
        cp.wait()

        y = p_blk[...] + stage[...] + r_blk[...]
        rms = jnp.sqrt(jnp.mean(y * y, axis=-1, keepdims=True) + EPS)
        o_blk[...] = y / rms * g_blk[...]

        @pl.when(k == K - 1)
        def _():
            for c in range(C):
                y_rdma(c).wait_send()
                fwd_rdma(c).wait_send()

    return pl.pallas_call(
        body,
        grid=(K,),
        in_specs=[
            pl.BlockSpec(memory_space=pl.ANY),
            pl.BlockSpec((ROWS, D), lambda i: (i, 0)),
            pl.BlockSpec((ROWS, D), lambda i: (i, 0)),
            pl.BlockSpec((1, D), lambda i: (0, 0)),
        ],
        out_specs=[
            pl.BlockSpec((ROWS, D), lambda i: (i, 0)),
            pl.BlockSpec(memory_space=pl.ANY),
        ],
        out_shape=[
            jax.ShapeDtypeStruct((M, D), jnp.float32),
            jax.ShapeDtypeStruct((M, D), jnp.float32),
        ],
        scratch_shapes=[
            pltpu.VMEM((ROWS, D), jnp.float32),
            pltpu.SemaphoreType.DMA((C,)),
            pltpu.SemaphoreType.DMA((C,)),
            pltpu.SemaphoreType.DMA((C,)),
            pltpu.SemaphoreType.DMA((C,)),
            pltpu.SemaphoreType.DMA,
        ],
        compiler_params=pltpu.CompilerParams(
            collective_id=0, vmem_limit_bytes=60 * 1024 * 1024
        ),
    )(partial, p2d, resid, gamma2d)[0]


# device time: 508479 ns/iter; 1.0278x vs baseline; 1.0278x over previous
import jax
import jax.numpy as jnp
from jax import lax
from jax.experimental import pallas as pl
from jax.experimental.pallas import tpu as pltpu

M, D = 8192, 2048
EPS = 1e-6

K = 32
C = K // 2
ROWS = M // K
HALF = M // 2


def kernel(partial, resid, gamma):
    gamma2d = gamma.reshape(1, D)

    def body(
        p_any,
        p_blk,
        r_blk,
        g_blk,
        o_blk,
        recv_any,
        stage,
        y_send_sems,
        y_recv_sems,
        x_send_sems,
        x_recv_sems,
        copy_sem,
    ):
        k = pl.program_id(0)
        my_x = lax.axis_index("x")
        my_y = lax.axis_index("y")
        my_z = lax.axis_index("z")
        ynbr = (my_x, 1 - my_y, my_z)
        xnbr = (1 - my_x, my_y, my_z)
        half0 = my_x * HALF

        def y_rdma(c):
            return pltpu.make_async_remote_copy(
                src_ref=p_any.at[0, pl.ds(half0 + c * ROWS, ROWS), :],
                dst_ref=recv_any.at[pl.ds(half0 + c * ROWS, ROWS), :],
                send_sem=y_send_sems.at[c],
                recv_sem=y_recv_sems.at[c],
                device_id=ynbr,
                device_id_type=pl.DeviceIdType.MESH,
            )

        def fwd_rdma(c):
            return pltpu.make_async_remote_copy(
                src_ref=recv_any.at[pl.ds(half0 + c * ROWS, ROWS), :],
                dst_ref=recv_any.at[pl.ds(half0 + c * ROWS, ROWS), :],
                send_sem=x_send_sems.at[c],
                recv_sem=x_recv_sems.at[c],
                device_id=xnbr,
                device_id_type=pl.DeviceIdType.MESH,
            )

        @pl.when(k == 0)
        def _():
            barrier = pltpu.get_barrier_semaphore()
            for nbr in (ynbr, xnbr):
                pl.semaphore_signal(
                    barrier, inc=1, device_id=nbr,
                    device_id_type=pl.DeviceIdType.MESH,
                )
            pl.semaphore_wait(barrier, 2)
            for c in range(C):
                y_rdma(c).start()

        @pl.when(k < C)
        def _():
            pltpu.make_async_remote_copy(
                src_ref=p_any.at[0, pl.ds(0, ROWS), :],
                dst_ref=recv_any.at[pl.ds(half0 + k * ROWS, ROWS), :],
                send_sem=y_send_sems.at[0],
                recv_sem=y_recv_sems.at[k],
                device_id=ynbr,
                device_id_type=pl.DeviceIdType.MESH,
            ).wait_recv()
            fwd_rdma(k).start()

        @pl.when(((my_x == 0) & (k >= C)) | ((my_x == 1) & (k < C)))
        def _():
            cc = k % C
            pltpu.make_async_remote_copy(
                src_ref=recv_any.at[pl.ds(0, ROWS), :],
                dst_ref=recv_any.at[
                    pl.ds((1 - my_x) * HALF + cc * ROWS, ROWS), :
                ],
                send_sem=x_send_sems.at[0],
                recv_sem=x_recv_sems.at[cc],
                device_id=xnbr,
                device_id_type=pl.DeviceIdType.MESH,
            ).wait_recv()

        cp = pltpu.make_async_copy(
            recv_any.at[pl.ds(k * ROWS, ROWS), :], stage, copy_sem
        )
        cp.start()
        cp.wait()

        y = p_blk[0] + stage[...] + r_blk[...]
        rms = jnp.sqrt(jnp.mean(y * y, axis=-1, keepdims=True) + EPS)
        o_blk[...] = y / rms * g_blk[...]

        @pl.when(k == K - 1)
        def _():
            for c in range(C):
                y_rdma(c).wait_send()
                fwd_rdma(c).wait_send()

    return pl.pallas_call(
        body,
        grid=(K,),
        in_specs=[
            pl.BlockSpec(memory_space=pl.ANY),
            pl.BlockSpec((1, ROWS, D), lambda i: (0, i, 0)),
            pl.BlockSpec((ROWS, D), lambda i: (i, 0)),
            pl.BlockSpec((1, D), lambda i: (0, 0)),
        ],
        out_specs=[
            pl.BlockSpec((ROWS, D), lambda i: (i, 0)),
            pl.BlockSpec(memory_space=pl.ANY),
        ],
        out_shape=[
            jax.ShapeDtypeStruct((M, D), jnp.float32),
            jax.ShapeDtypeStruct((M, D), jnp.float32),
        ],
        scratch_shapes=[
            pltpu.VMEM((ROWS, D), jnp.float32),
            pltpu.SemaphoreType.DMA((C,)),
            pltpu.SemaphoreType.DMA((C,)),
            pltpu.SemaphoreType.DMA((C,)),
            pltpu.SemaphoreType.DMA((C,)),
            pltpu.SemaphoreType.DMA,
        ],
        compiler_params=pltpu.CompilerParams(
            collective_id=0, vmem_limit_bytes=60 * 1024 * 1024
        ),
    )(partial, partial, resid, gamma2d)[0]


# device time: 389209 ns/iter; 1.3428x vs baseline; 1.3064x over previous
import jax
import jax.numpy as jnp
from jax import lax
from jax.experimental import pallas as pl
from jax.experimental.pallas import tpu as pltpu

M, D = 8192, 2048
EPS = 1e-6

K = 32
ROWS = M // K
QC = 8
QROWS = QC * ROWS


def kernel(partial, resid, gamma):
    gamma2d = gamma.reshape(1, D)

    xb = lax.axis_index("x")
    zbit = lax.axis_index("z") % 2
    order = jnp.stack(
        [
            2 * xb + zbit,
            2 * xb + (1 - zbit),
            2 * (1 - xb) + zbit,
            2 * (1 - xb) + (1 - zbit),
        ]
    ).astype(jnp.int32)
    s = jnp.arange(K, dtype=jnp.int32)
    blocks = order[s // QC] * QC + (s % QC)

    def body(
        b_ref,
        p_any,
        p_blk,
        r_blk,
        g_blk,
        o_blk,
        recv_any,
        stage,
        y_send, y_recv,
        x1_send, x1_recv,
        z1_send, z1_recv,
        x2_send, x2_recv,
        z2_send, z2_recv,
        copy_sem,
    ):
        k = pl.program_id(0)
        my_x = lax.axis_index("x")
        my_y = lax.axis_index("y")
        my_z = lax.axis_index("z")
        zb = my_z % 2
        ynbr = (my_x, 1 - my_y, my_z)
        xnbr = (1 - my_x, my_y, my_z)
        znbr = (my_x, my_y, my_z + 1 - 2 * zb)

        qy0 = (2 * my_x + zb) * QROWS
        qz0 = (2 * my_x + (1 - zb)) * QROWS
        qx0 = (2 * (1 - my_x) + zb) * QROWS

        def rdma(row0, c, send_sem, recv_sem, dev, src=None):
            src = recv_any if src is None else src
            return pltpu.make_async_remote_copy(
                src_ref=src.at[pl.ds(row0 + c * ROWS, ROWS), :],
                dst_ref=recv_any.at[pl.ds(row0 + c * ROWS, ROWS), :],
                send_sem=send_sem.at[c],
                recv_sem=recv_sem.at[c],
                device_id=dev,
                device_id_type=pl.DeviceIdType.MESH,
            )

        def y_rdma(c):
            return pltpu.make_async_remote_copy(
                src_ref=p_any.at[0, pl.ds(qy0 + c * ROWS, ROWS), :],
                dst_ref=recv_any.at[pl.ds(qy0 + c * ROWS, ROWS), :],
                send_sem=y_send.at[c],
                recv_sem=y_recv.at[c],
                device_id=ynbr,
                device_id_type=pl.DeviceIdType.MESH,
            )

        @pl.when(k == 0)
        def _():
            barrier = pltpu.get_barrier_semaphore()
            for nbr in (ynbr, xnbr, znbr):
                pl.semaphore_signal(
                    barrier, inc=1, device_id=nbr,
                    device_id_type=pl.DeviceIdType.MESH,
                )
            pl.semaphore_wait(barrier, 3)
            for c in range(QC):
                y_rdma(c).start()

        @pl.when(k < QC)
        def _():
            y_rdma_dyn = pltpu.make_async_remote_copy(
                src_ref=p_any.at[0, pl.ds(0, ROWS), :],
                dst_ref=recv_any.at[pl.ds(qy0 + k * ROWS, ROWS), :],
                send_sem=y_send.at[0],
                recv_sem=y_recv.at[k],
                device_id=ynbr,
                device_id_type=pl.DeviceIdType.MESH,
            )
            y_rdma_dyn.wait_recv()
            rdma(qy0, k, x1_send, x1_recv, xnbr).start()
            rdma(qy0, k, z1_send, z1_recv, znbr).start()

        @pl.when((k >= QC) & (k < 2 * QC))
        def _():
            c = k - QC
            rdma(qz0, c, z1_send, z1_recv, znbr).wait_recv()

            @pl.when(c < QC // 2)
            def _():
                rdma(qz0, c, x2_send, x2_recv, xnbr).start()

        @pl.when((k >= 2 * QC) & (k < 3 * QC))
        def _():
            c = k - 2 * QC
            rdma(qx0, c, x1_send, x1_recv, xnbr).wait_recv()

            @pl.when(c >= QC // 2)
            def _():
                rdma(
                    qx0 + (QC // 2) * ROWS, c - QC // 2,
                    z2_send, z2_recv, znbr,
                ).start()

        @pl.when(k >= 3 * QC)
        def _():
            c = k - 3 * QC

            @pl.when(c < QC // 2)
            def _():
                rdma(qz0, c, x2_send, x2_recv, xnbr).wait_recv()

            @pl.when(c >= QC // 2)
            def _():
                rdma(qx0, c - QC // 2, z2_send, z2_recv, znbr).wait_recv()

        b = b_ref[k]
        cp = pltpu.make_async_copy(
            recv_any.at[pl.ds(b * ROWS, ROWS), :], stage, copy_sem
        )
        cp.start()
        cp.wait()

        y = p_blk[0] + stage[...] + r_blk[...]
        rms = jnp.sqrt(jnp.mean(y * y, axis=-1, keepdims=True) + EPS)
        o_blk[...] = y / rms * g_blk[...]

        @pl.when(k == K - 1)
        def _():
            for c in range(QC):
                y_rdma(c).wait_send()
                rdma(qy0, c, x1_send, x1_recv, xnbr).wait_send()
                rdma(qy0, c, z1_send, z1_recv, znbr).wait_send()
            for c in range(QC // 2):
                rdma(qz0, c, x2_send, x2_recv, xnbr).wait_send()
                rdma(qx0, c, z2_send, z2_recv, znbr).wait_send()

    grid_spec = pltpu.PrefetchScalarGridSpec(
        num_scalar_prefetch=1,
        grid=(K,),
        in_specs=[
            pl.BlockSpec(memory_space=pl.ANY),
            pl.BlockSpec((1, ROWS, D), lambda i, b: (0, b[i], 0)),
            pl.BlockSpec((ROWS, D), lambda i, b: (b[i], 0)),
            pl.BlockSpec((1, D), lambda i, b: (0, 0)),
        ],
        out_specs=[
            pl.BlockSpec((ROWS, D), lambda i, b: (b[i], 0)),
            pl.BlockSpec(memory_space=pl.ANY),
        ],
        scratch_shapes=[
            pltpu.VMEM((ROWS, D), jnp.float32),
            pltpu.SemaphoreType.DMA((QC,)),
            pltpu.SemaphoreType.DMA((QC,)),
            pltpu.SemaphoreType.DMA((QC,)),
            pltpu.SemaphoreType.DMA((QC,)),
            pltpu.SemaphoreType.DMA((QC,)),
            pltpu.SemaphoreType.DMA((QC,)),
            pltpu.SemaphoreType.DMA((QC // 2,)),
            pltpu.SemaphoreType.DMA((QC // 2,)),
            pltpu.SemaphoreType.DMA((QC // 2,)),
            pltpu.SemaphoreType.DMA((QC // 2,)),
            pltpu.SemaphoreType.DMA,
        ],
    )

    return pl.pallas_call(
        body,
        grid_spec=grid_spec,
        out_shape=[
            jax.ShapeDtypeStruct((M, D), jnp.float32),
            jax.ShapeDtypeStruct((M, D), jnp.float32),
        ],
        compiler_params=pltpu.CompilerParams(
            collective_id=0, vmem_limit_bytes=60 * 1024 * 1024
        ),
    )(blocks, partial, partial, resid, gamma2d)[0]


# device time: 358159 ns/iter; 1.4592x vs baseline; 1.0867x over previous
import jax
import jax.numpy as jnp
from jax import lax
from jax.experimental import pallas as pl
from jax.experimental.pallas import tpu as pltpu

M, D = 8192, 2048
EPS = 1e-6

K = 32
ROWS = M // K
QC = 8
QROWS = QC * ROWS


def kernel(partial, resid, gamma):
    gamma2d = gamma.reshape(1, D)

    xb = lax.axis_index("x")
    zbit = lax.axis_index("z") % 2
    order = jnp.stack(
        [
            2 * xb + zbit,
            2 * xb + (1 - zbit),
            2 * (1 - xb) + zbit,
            2 * (1 - xb) + (1 - zbit),
        ]
    ).astype(jnp.int32)
    s = jnp.arange(K, dtype=jnp.int32)
    blocks = order[s // QC] * QC + (s % QC)

    def body(
        b_ref,
        p_any,
        p_blk,
        r_blk,
        g_blk,
        o_blk,
        recv_any,
        stage,
        y_send, y_recv,
        x1_send, x1_recv,
        z1_send, z1_recv,
        x2_send, x2_recv,
        z2_send, z2_recv,
        copy_sem,
    ):
        k = pl.program_id(0)
        my_x = lax.axis_index("x")
        my_y = lax.axis_index("y")
        my_z = lax.axis_index("z")
        zb = my_z % 2
        ynbr = (my_x, 1 - my_y, my_z)
        xnbr = (1 - my_x, my_y, my_z)
        znbr = (my_x, my_y, my_z + 1 - 2 * zb)

        qy0 = (2 * my_x + zb) * QROWS
        qz0 = (2 * my_x + (1 - zb)) * QROWS
        qx0 = (2 * (1 - my_x) + zb) * QROWS
        qd0 = (2 * (1 - my_x) + (1 - zb)) * QROWS

        def rdma(row0, c, send_sem, recv_sem, dev, src=None):
            src = recv_any if src is None else src
            return pltpu.make_async_remote_copy(
                src_ref=src.at[pl.ds(row0 + c * ROWS, ROWS), :],
                dst_ref=recv_any.at[pl.ds(row0 + c * ROWS, ROWS), :],
                send_sem=send_sem.at[c],
                recv_sem=recv_sem.at[c],
                device_id=dev,
                device_id_type=pl.DeviceIdType.MESH,
            )

        def y_rdma(c):
            row0 = qy0 + c * ROWS if c < QC else qd0 + (c - QC) * ROWS
            return pltpu.make_async_remote_copy(
                src_ref=p_any.at[0, pl.ds(row0, ROWS), :],
                dst_ref=recv_any.at[pl.ds(row0, ROWS), :],
                send_sem=y_send.at[c],
                recv_sem=y_recv.at[c],
                device_id=ynbr,
                device_id_type=pl.DeviceIdType.MESH,
            )

        @pl.when(k == 0)
        def _():
            barrier = pltpu.get_barrier_semaphore()
            for nbr in (ynbr, xnbr, znbr):
                pl.semaphore_signal(
                    barrier, inc=1, device_id=nbr,
                    device_id_type=pl.DeviceIdType.MESH,
                )
            pl.semaphore_wait(barrier, 3)
            for c in range(QC + 3):
                y_rdma(c).start()

        @pl.when(k < QC)
        def _():
            y_rdma_dyn = pltpu.make_async_remote_copy(
                src_ref=p_any.at[0, pl.ds(0, ROWS), :],
                dst_ref=recv_any.at[pl.ds(qy0 + k * ROWS, ROWS), :],
                send_sem=y_send.at[0],
                recv_sem=y_recv.at[k],
                device_id=ynbr,
                device_id_type=pl.DeviceIdType.MESH,
            )
            y_rdma_dyn.wait_recv()
            rdma(qy0, k, x1_send, x1_recv, xnbr).start()
            rdma(qy0, k, z1_send, z1_recv, znbr).start()

        @pl.when((k >= QC) & (k < 2 * QC))
        def _():
            c = k - QC
            rdma(qz0, c, z1_send, z1_recv, znbr).wait_recv()

            @pl.when((c >= 3) & (c < 6))
            def _():
                rdma(qz0 + 3 * ROWS, c - 3, x2_send, x2_recv, xnbr).start()

        @pl.when((k >= 2 * QC) & (k < 3 * QC))
        def _():
            c = k - 2 * QC
            rdma(qx0, c, x1_send, x1_recv, xnbr).wait_recv()

            @pl.when(c >= 6)
            def _():
                rdma(qx0 + 6 * ROWS, c - 6, z2_send, z2_recv, znbr).start()

        @pl.when(k >= 3 * QC)
        def _():
            c = k - 3 * QC

            @pl.when(c < 3)
            def _():
                pltpu.make_async_remote_copy(
                    src_ref=p_any.at[0, pl.ds(0, ROWS), :],
                    dst_ref=recv_any.at[pl.ds(qd0 + c * ROWS, ROWS), :],
                    send_sem=y_send.at[0],
                    recv_sem=y_recv.at[QC + c],
                    device_id=ynbr,
                    device_id_type=pl.DeviceIdType.MESH,
                ).wait_recv()

            @pl.when((c >= 3) & (c < 6))
            def _():
                rdma(qz0 + 3 * ROWS, c - 3, x2_send, x2_recv, xnbr).wait_recv()

            @pl.when(c >= 6)
            def _():
                rdma(qx0 + 6 * ROWS, c - 6, z2_send, z2_recv, znbr).wait_recv()

        b = b_ref[k]
        cp = pltpu.make_async_copy(
            recv_any.at[pl.ds(b * ROWS, ROWS), :], stage, copy_sem
        )
        cp.start()
        cp.wait()

        y = p_blk[0] + stage[...] + r_blk[...]
        rms = jnp.sqrt(jnp.mean(y * y, axis=-1, keepdims=True) + EPS)
        o_blk[...] = y / rms * g_blk[...]

        @pl.when(k == K - 1)
        def _():
            for c in range(QC + 3):
                y_rdma(c).wait_send()
            for c in range(QC):
                rdma(qy0, c, x1_send, x1_recv, xnbr).wait_send()
                rdma(qy0, c, z1_send, z1_recv, znbr).wait_send()
            for c in range(3):
                rdma(qz0 + 3 * ROWS, c, x2_send, x2_recv, xnbr).wait_send()
            for c in range(2):
                rdma(qx0 + 6 * ROWS, c, z2_send, z2_recv, znbr).wait_send()

    grid_spec = pltpu.PrefetchScalarGridSpec(
        num_scalar_prefetch=1,
        grid=(K,),
        in_specs=[
            pl.BlockSpec(memory_space=pl.ANY),
            pl.BlockSpec((1, ROWS, D), lambda i, b: (0, b[i], 0)),
            pl.BlockSpec((ROWS, D), lambda i, b: (b[i], 0)),
            pl.BlockSpec((1, D), lambda i, b: (0, 0)),
        ],
        out_specs=[
            pl.BlockSpec((ROWS, D), lambda i, b: (b[i], 0)),
            pl.BlockSpec(memory_space=pl.ANY),
        ],
        scratch_shapes=[
            pltpu.VMEM((ROWS, D), jnp.float32),
            pltpu.SemaphoreType.DMA((QC + 3,)),
            pltpu.SemaphoreType.DMA((QC + 3,)),
            pltpu.SemaphoreType.DMA((QC,)),
            pltpu.SemaphoreType.DMA((QC,)),
            pltpu.SemaphoreType.DMA((QC,)),
            pltpu.SemaphoreType.DMA((QC,)),
            pltpu.SemaphoreType.DMA((3,)),
            pltpu.SemaphoreType.DMA((3,)),
            pltpu.SemaphoreType.DMA((2,)),
            pltpu.SemaphoreType.DMA((2,)),
            pltpu.SemaphoreType.DMA,
        ],
    )

    return pl.pallas_call(
        body,
        grid_spec=grid_spec,
        out_shape=[
            jax.ShapeDtypeStruct((M, D), jnp.float32),
            jax.ShapeDtypeStruct((M, D), jnp.float32),
        ],
        compiler_params=pltpu.CompilerParams(
            collective_id=0, vmem_limit_bytes=60 * 1024 * 1024
        ),
    )(blocks, partial, partial, resid, gamma2d)[0]
